# baseline (speedup 1.0000x reference)
import functools
import jax
import jax.numpy as jnp
from jax import lax
from jax.experimental import pallas as pl
from jax.experimental.pallas import tpu as pltpu
from jax.experimental.pallas import tpu_sc as plsc

N_NODES = 10000
D = 128
E = 320000
N_TILES = 32
CHUNK = 16
CHUNKS_PER_TILE = 640
E_PAD = N_TILES * CHUNKS_PER_TILE * CHUNK
SUBS = 16
ROWS_PER_SUB = 624
TAIL_ROWS = N_NODES - SUBS * ROWS_PER_SUB


def _gate_body(x_ref, wg_ref, b_ref, out_ref):
    x = x_ref[...]
    pre = jnp.sum(x * wg_ref[...], axis=1, keepdims=True) + b_ref[0, 0]
    out_ref[...] = x * jax.nn.sigmoid(pre)


def _final_body(p_ref, w_ref, out_ref):
    s = p_ref[0] + p_ref[1]
    y = lax.dot_general(s, w_ref[...], (((1,), (0,)), ((), ())),
                        preferred_element_type=jnp.float32)
    out_ref[...] = jnp.maximum(y, 0.0)


def _agg_body(idx_hbm, w_hbm, tab_hbm, p_hbm,
              gin0, gin1, gout0, gout1, idxb, wb, tab, acc,
              sg0, sg1, ss0, ss1, si0, si1, si2, si3):
    gin = (gin0, gin1)
    gout = (gout0, gout1)
    sem_g = (sg0, sg1)
    sem_s = (ss0, ss1)
    sem_i = (si0, si1, si2, si3)
    cid = lax.axis_index("c")
    sid = lax.axis_index("s")
    wid = cid * SUBS + sid

    @pl.when(sid < 9)
    def _():
        pltpu.sync_copy(tab_hbm.at[pl.ds(sid * 512, 512)],
                        tab.at[pl.ds(sid * 512, 512)])

    @pl.when(sid == 9)
    def _():
        pltpu.sync_copy(tab_hbm.at[pl.ds(4608, 392)],
                        tab.at[pl.ds(4608, 392)])

    zero16 = jnp.zeros((16,), jnp.float32)

    def zloop(i, carry):
        gout0[i // 8, pl.ds((i % 8) * 16, 16)] = zero16
        return carry
    lax.fori_loop(0, CHUNK * 8, zloop, 0)

    def zacc(kk, carry):
        pltpu.async_copy(gout0,
                         acc.at[pl.ds(sid * ROWS_PER_SUB + kk * CHUNK, CHUNK)],
                         sg0)
        return carry
    lax.fori_loop(0, ROWS_PER_SUB // CHUNK, zacc, 0)

    @pl.when(sid == 0)
    def _():
        pltpu.sync_copy(gout0, acc.at[pl.ds(SUBS * ROWS_PER_SUB, TAIL_ROWS)])

    def zdrain(kk, carry):
        pltpu.make_async_copy(gout0, acc.at[pl.ds(sid * ROWS_PER_SUB, CHUNK)],
                              sg0).wait()
        return carry
    lax.fori_loop(0, ROWS_PER_SUB // CHUNK, zdrain, 0)

    plsc.subcore_barrier()

    def _fetch_idx(c, slot):
        pltpu.async_copy(idx_hbm.at[wid, c], idxb.at[slot], sem_i[slot])
        pltpu.async_copy(w_hbm.at[wid, c], wb.at[slot], sem_i[slot])

    def _wait_idx(c, slot):
        pltpu.make_async_copy(idx_hbm.at[wid, c], idxb.at[slot],
                              sem_i[slot]).wait()
        pltpu.make_async_copy(w_hbm.at[wid, c], wb.at[slot],
                              sem_i[slot]).wait()

    def _scale(gi, go, slot):
        wvec = wb[slot, pl.ds(0, 16)]
        bvec = idxb[slot, 1, pl.ds(0, 16)]
        for e in range(CHUNK):
            ws = wvec[e]
            base = bvec[e]
            ins = [gi[e, pl.ds(base + 16 * jj, 16)] for jj in range(4)]
            for jj in range(4):
                a = lax.bitcast_convert_type(ins[jj] << 16, jnp.float32)
                b = lax.bitcast_convert_type(ins[jj] & -65536, jnp.float32)
                go[e, pl.ds(32 * jj, 16)] = a * ws
                go[e, pl.ds(32 * jj + 16, 16)] = b * ws

    _fetch_idx(0, 0)
    _fetch_idx(1, 1)
    _wait_idx(0, 0)
    pltpu.async_copy(tab.at[idxb.at[0, 0]], gin0, sg0)

    def pipe_body(i, carry):
        for k in range(4):
            c = 4 * i + k
            r = k % 2
            rn = (k + 1) % 2
            k2 = (k + 2) % 4
            kn = (k + 1) % 4

            if k < 2:
                _fetch_idx(c + 2, k2)
            else:
                @pl.when(i < (CHUNKS_PER_TILE // 4) - 1)
                def _():
                    _fetch_idx(c + 2, k2)

            if k < 3:
                _wait_idx(c + 1, kn)
            else:
                @pl.when(i < (CHUNKS_PER_TILE // 4) - 1)
                def _():
                    _wait_idx(c + 1, kn)

            drain = pltpu.make_async_copy(gout[rn], acc.at[idxb.at[kn, 2]],
                                          sem_s[rn])
            if k == 0:
                @pl.when(i >= 1)
                def _():
                    drain.wait()
            else:
                drain.wait()

            gath = pltpu.make_async_copy(tab.at[idxb.at[kn, 0]], gin[rn],
                                         sem_g[rn])
            if k < 3:
                gath.start()
            else:
                @pl.when(i < (CHUNKS_PER_TILE // 4) - 1)
                def _():
                    gath.start()

            pltpu.make_async_copy(tab.at[idxb.at[k, 0]], gin[r],
                                  sem_g[r]).wait()
            _scale(gin[r], gout[r], k)
            pltpu.async_copy(gout[r], acc.at[idxb.at[k, 2]], sem_s[r],
                             add=True)
        return carry
    lax.fori_loop(0, CHUNKS_PER_TILE // 4, pipe_body, 0)

    pltpu.make_async_copy(gout1, acc.at[idxb.at[3, 2]], ss1).wait()

    plsc.subcore_barrier()

    pltpu.sync_copy(acc.at[pl.ds(sid * ROWS_PER_SUB, ROWS_PER_SUB)],
                    p_hbm.at[cid, pl.ds(sid * ROWS_PER_SUB, ROWS_PER_SUB)])

    @pl.when(sid == 0)
    def _():
        pltpu.sync_copy(acc.at[pl.ds(SUBS * ROWS_PER_SUB, TAIL_ROWS)],
                        p_hbm.at[cid, pl.ds(SUBS * ROWS_PER_SUB, TAIL_ROWS)])


_agg = functools.partial(
    pl.kernel,
    mesh=plsc.VectorSubcoreMesh(core_axis_name="c", subcore_axis_name="s"),
    out_type=jax.ShapeDtypeStruct((2, N_NODES, D), jnp.float32),
    scratch_types=[
        pltpu.VMEM((CHUNK, D), jnp.int32),
        pltpu.VMEM((CHUNK, D), jnp.int32),
        pltpu.VMEM((CHUNK, D), jnp.float32),
        pltpu.VMEM((CHUNK, D), jnp.float32),
        pltpu.VMEM((4, 3, CHUNK), jnp.int32),
        pltpu.VMEM((4, CHUNK), jnp.float32),
        pltpu.VMEM_SHARED((N_NODES // 2, D), jnp.int32),
        pltpu.VMEM_SHARED((N_NODES, D), jnp.float32),
        pltpu.SemaphoreType.DMA,
        pltpu.SemaphoreType.DMA,
        pltpu.SemaphoreType.DMA,
        pltpu.SemaphoreType.DMA,
        pltpu.SemaphoreType.DMA,
        pltpu.SemaphoreType.DMA,
        pltpu.SemaphoreType.DMA,
        pltpu.SemaphoreType.DMA,
    ],
)(_agg_body)


BLK = 2000


@jax.jit
def kernel(x, edge_index, edge_weight, W, W_gate, b_gate):
    dst = edge_index[0].astype(jnp.int32)
    src = edge_index[1].astype(jnp.int32)
    pad = E_PAD - E
    src = jnp.concatenate([src, jnp.zeros((pad,), jnp.int32)])
    dst = jnp.concatenate([dst, jnp.zeros((pad,), jnp.int32)])
    w = jnp.concatenate([edge_weight, jnp.zeros((pad,), jnp.float32)])
    idx = jnp.stack([src >> 1, (src & 1) * 64, dst])
    idx4 = idx.reshape(3, N_TILES, CHUNKS_PER_TILE, CHUNK).transpose(1, 2, 0, 3)
    w3 = w.reshape(N_TILES, CHUNKS_PER_TILE, CHUNK)

    xg = pl.pallas_call(
        _gate_body,
        grid=(N_NODES // BLK,),
        in_specs=[
            pl.BlockSpec((BLK, D), lambda i: (i, 0)),
            pl.BlockSpec((1, D), lambda i: (0, 0)),
            pl.BlockSpec((1, 1), lambda i: (0, 0)),
        ],
        out_specs=pl.BlockSpec((BLK, D), lambda i: (i, 0)),
        out_shape=jax.ShapeDtypeStruct((N_NODES, D), jnp.float32),
    )(x, W_gate.reshape(1, D), b_gate.reshape(1, 1))

    xu = lax.bitcast_convert_type(xg.astype(jnp.bfloat16),
                                  jnp.uint16).astype(jnp.uint32)
    xu = xu.reshape(N_NODES, 4, 2, 16)
    words = (xu[:, :, 0, :] | (xu[:, :, 1, :] << 16)).astype(jnp.uint32)
    tab = words.reshape(N_NODES // 2, D).astype(jnp.int32)

    p = _agg(idx4, w3, tab)

    out = pl.pallas_call(
        _final_body,
        grid=(N_NODES // BLK,),
        in_specs=[
            pl.BlockSpec((2, BLK, D), lambda i: (0, i, 0)),
            pl.BlockSpec((D, D), lambda i: (0, 0)),
        ],
        out_specs=pl.BlockSpec((BLK, D), lambda i: (i, 0)),
        out_shape=jax.ShapeDtypeStruct((N_NODES, D), jnp.float32),
    )(p, W)
    return out


# speedup vs baseline: 1.1968x; 1.1968x over previous
import functools
import jax
import jax.numpy as jnp
from jax import lax
from jax.experimental import pallas as pl
from jax.experimental.pallas import tpu as pltpu
from jax.experimental.pallas import tpu_sc as plsc

N_NODES = 10000
D = 128
E = 320000
N_TILES = 32
CHUNK = 16
CHUNKS_PER_TILE = 640
E_PAD = N_TILES * CHUNKS_PER_TILE * CHUNK
SUBS = 16
ROWS_PER_SUB = 624
TAIL_ROWS = N_NODES - SUBS * ROWS_PER_SUB


def _gate_body(x_ref, wg_ref, b_ref, out_ref):
    x = x_ref[...]
    pre = jnp.sum(x * wg_ref[...], axis=1, keepdims=True) + b_ref[0, 0]
    out_ref[...] = x * jax.nn.sigmoid(pre)


def _final_body(p_ref, w_ref, out_ref):
    s = p_ref[0] + p_ref[1]
    y = lax.dot_general(s, w_ref[...], (((1,), (0,)), ((), ())),
                        preferred_element_type=jnp.float32)
    out_ref[...] = jnp.maximum(y, 0.0)


def _agg_body(srow_hbm, base_hbm, dst_hbm, w_hbm, tab_hbm, p_hbm,
              gin0, gin1, gout0, gout1, rowb, baseb, dstb, wb, tab, acc,
              sg0, sg1, ss0, ss1, si0, si1, si2, si3):
    gin = (gin0, gin1)
    gout = (gout0, gout1)
    sem_g = (sg0, sg1)
    sem_s = (ss0, ss1)
    sem_i = (si0, si1, si2, si3)
    cid = lax.axis_index("c")
    sid = lax.axis_index("s")
    wid = cid * SUBS + sid

    @pl.when(sid < 9)
    def _():
        pltpu.sync_copy(tab_hbm.at[pl.ds(sid * 512, 512)],
                        tab.at[pl.ds(sid * 512, 512)])

    @pl.when(sid == 9)
    def _():
        pltpu.sync_copy(tab_hbm.at[pl.ds(4608, 392)],
                        tab.at[pl.ds(4608, 392)])

    zero16 = jnp.zeros((16,), jnp.float32)

    def zloop(i, carry):
        gout0[i // 8, pl.ds((i % 8) * 16, 16)] = zero16
        return carry
    lax.fori_loop(0, CHUNK * 8, zloop, 0)

    def zacc(kk, carry):
        pltpu.async_copy(gout0,
                         acc.at[pl.ds(sid * ROWS_PER_SUB + kk * CHUNK, CHUNK)],
                         sg0)
        return carry
    lax.fori_loop(0, ROWS_PER_SUB // CHUNK, zacc, 0)

    @pl.when(sid == 0)
    def _():
        pltpu.sync_copy(gout0, acc.at[pl.ds(SUBS * ROWS_PER_SUB, TAIL_ROWS)])

    def zdrain(kk, carry):
        pltpu.make_async_copy(gout0, acc.at[pl.ds(sid * ROWS_PER_SUB, CHUNK)],
                              sg0).wait()
        return carry
    lax.fori_loop(0, ROWS_PER_SUB // CHUNK, zdrain, 0)

    plsc.subcore_barrier()

    def _fetch_idx(c, slot):
        pltpu.async_copy(srow_hbm.at[wid, c], rowb.at[slot], sem_i[slot])
        pltpu.async_copy(base_hbm.at[wid, c], baseb.at[slot], sem_i[slot])
        pltpu.async_copy(dst_hbm.at[wid, c], dstb.at[slot], sem_i[slot])
        pltpu.async_copy(w_hbm.at[wid, c], wb.at[slot], sem_i[slot])

    def _wait_idx(c, slot):
        pltpu.make_async_copy(srow_hbm.at[wid, c], rowb.at[slot],
                              sem_i[slot]).wait()
        pltpu.make_async_copy(base_hbm.at[wid, c], baseb.at[slot],
                              sem_i[slot]).wait()
        pltpu.make_async_copy(dst_hbm.at[wid, c], dstb.at[slot],
                              sem_i[slot]).wait()
        pltpu.make_async_copy(w_hbm.at[wid, c], wb.at[slot],
                              sem_i[slot]).wait()

    def _scale(gi, go, slot):
        wvec = wb[slot, pl.ds(0, 16)]
        bvec = baseb[slot, pl.ds(0, 16)]
        for e in range(CHUNK):
            ws = wvec[e]
            base = bvec[e]
            ins = [gi[e, pl.ds(base + 16 * jj, 16)] for jj in range(4)]
            for jj in range(4):
                a = lax.bitcast_convert_type(ins[jj] << 16, jnp.float32)
                b = lax.bitcast_convert_type(ins[jj] & -65536, jnp.float32)
                go[e, pl.ds(16 * jj, 16)] = a * ws
                go[e, pl.ds(64 + 16 * jj, 16)] = b * ws

    _fetch_idx(0, 0)
    _fetch_idx(1, 1)
    _wait_idx(0, 0)
    pltpu.async_copy(tab.at[rowb.at[0]], gin0, sg0)

    def pipe_body(i, carry):
        for k in range(4):
            c = 4 * i + k
            r = k % 2
            rn = (k + 1) % 2
            k2 = (k + 2) % 4
            kn = (k + 1) % 4

            if k < 2:
                _fetch_idx(c + 2, k2)
            else:
                @pl.when(i < (CHUNKS_PER_TILE // 4) - 1)
                def _():
                    _fetch_idx(c + 2, k2)

            if k < 3:
                _wait_idx(c + 1, kn)
            else:
                @pl.when(i < (CHUNKS_PER_TILE // 4) - 1)
                def _():
                    _wait_idx(c + 1, kn)

            drain = pltpu.make_async_copy(gout[rn], acc.at[dstb.at[kn]],
                                          sem_s[rn])
            if k == 0:
                @pl.when(i >= 1)
                def _():
                    drain.wait()
            else:
                drain.wait()

            gath = pltpu.make_async_copy(tab.at[rowb.at[kn]], gin[rn],
                                         sem_g[rn])
            if k < 3:
                gath.start()
            else:
                @pl.when(i < (CHUNKS_PER_TILE // 4) - 1)
                def _():
                    gath.start()

            pltpu.make_async_copy(tab.at[rowb.at[k]], gin[r],
                                  sem_g[r]).wait()
            _scale(gin[r], gout[r], k)
            pltpu.async_copy(gout[r], acc.at[dstb.at[k]], sem_s[r],
                             add=True)
        return carry
    lax.fori_loop(0, CHUNKS_PER_TILE // 4, pipe_body, 0)

    pltpu.make_async_copy(gout1, acc.at[dstb.at[3]], ss1).wait()

    plsc.subcore_barrier()

    pltpu.sync_copy(acc.at[pl.ds(sid * ROWS_PER_SUB, ROWS_PER_SUB)],
                    p_hbm.at[cid, pl.ds(sid * ROWS_PER_SUB, ROWS_PER_SUB)])

    @pl.when(sid == 0)
    def _():
        pltpu.sync_copy(acc.at[pl.ds(SUBS * ROWS_PER_SUB, TAIL_ROWS)],
                        p_hbm.at[cid, pl.ds(SUBS * ROWS_PER_SUB, TAIL_ROWS)])


_agg = functools.partial(
    pl.kernel,
    mesh=plsc.VectorSubcoreMesh(core_axis_name="c", subcore_axis_name="s"),
    out_type=jax.ShapeDtypeStruct((2, N_NODES, D), jnp.float32),
    scratch_types=[
        pltpu.VMEM((CHUNK, D), jnp.int32),
        pltpu.VMEM((CHUNK, D), jnp.int32),
        pltpu.VMEM((CHUNK, D), jnp.float32),
        pltpu.VMEM((CHUNK, D), jnp.float32),
        pltpu.VMEM((4, CHUNK), jnp.int32),
        pltpu.VMEM((4, CHUNK), jnp.int32),
        pltpu.VMEM((4, CHUNK), jnp.int32),
        pltpu.VMEM((4, CHUNK), jnp.float32),
        pltpu.VMEM_SHARED((N_NODES // 2, D), jnp.int32),
        pltpu.VMEM_SHARED((N_NODES, D), jnp.float32),
        pltpu.SemaphoreType.DMA,
        pltpu.SemaphoreType.DMA,
        pltpu.SemaphoreType.DMA,
        pltpu.SemaphoreType.DMA,
        pltpu.SemaphoreType.DMA,
        pltpu.SemaphoreType.DMA,
        pltpu.SemaphoreType.DMA,
        pltpu.SemaphoreType.DMA,
    ],
)(_agg_body)


BLK = 1000


@jax.jit
def kernel(x, edge_index, edge_weight, W, W_gate, b_gate):
    dst = edge_index[0].astype(jnp.int32)
    src = edge_index[1].astype(jnp.int32)
    pad = E_PAD - E
    src = jnp.concatenate([src, jnp.zeros((pad,), jnp.int32)])
    dst = jnp.concatenate([dst, jnp.zeros((pad,), jnp.int32)])
    w = jnp.concatenate([edge_weight, jnp.zeros((pad,), jnp.float32)])
    srow3 = (src >> 1).reshape(N_TILES, CHUNKS_PER_TILE, CHUNK)
    base3 = ((src & 1) * 64).reshape(N_TILES, CHUNKS_PER_TILE, CHUNK)
    dst3 = dst.reshape(N_TILES, CHUNKS_PER_TILE, CHUNK)
    w3 = w.reshape(N_TILES, CHUNKS_PER_TILE, CHUNK)

    xg = pl.pallas_call(
        _gate_body,
        grid=(N_NODES // BLK,),
        in_specs=[
            pl.BlockSpec((BLK, D), lambda i: (i, 0)),
            pl.BlockSpec((1, D), lambda i: (0, 0)),
            pl.BlockSpec((1, 1), lambda i: (0, 0)),
        ],
        out_specs=pl.BlockSpec((BLK, D), lambda i: (i, 0)),
        out_shape=jax.ShapeDtypeStruct((N_NODES, D), jnp.float32),
    )(x, W_gate.reshape(1, D), b_gate.reshape(1, 1))

    xu = lax.bitcast_convert_type(xg.astype(jnp.bfloat16),
                                  jnp.uint16).astype(jnp.int32)
    words = xu[:, :64] | (xu[:, 64:] << 16)
    tab = words.reshape(N_NODES // 2, D)

    p = _agg(srow3, base3, dst3, w3, tab)

    out = pl.pallas_call(
        _final_body,
        grid=(N_NODES // BLK,),
        in_specs=[
            pl.BlockSpec((2, BLK, D), lambda i: (0, i, 0)),
            pl.BlockSpec((D, D), lambda i: (0, 0)),
        ],
        out_specs=pl.BlockSpec((BLK, D), lambda i: (i, 0)),
        out_shape=jax.ShapeDtypeStruct((N_NODES, D), jnp.float32),
    )(p, W)
    return out
